# baseline (speedup 1.0000x reference)
import functools

import jax
import jax.numpy as jnp
from jax import lax
from jax.experimental import pallas as pl
from jax.experimental.pallas import tpu as pltpu
from jax.experimental.pallas import tpu_sc as plsc

F32 = jnp.float32
I32 = jnp.int32

NC = 2
NS = 16
NW = NC * NS
L = 16
SUB = 128
NDS = 10
CH = NDS * SUB

S_UPD_W1R0, S_UPD_W1R1, S_UPD_B1, S_UPD_W2C0, S_UPD_B2R0 = 0, 1, 2, 3, 4
NSLOT = 5
WP = NSLOT * L


def _slot(wv, s):
    return wv[pl.ds(s * L, L)]


def _scalars(vec):
    return [vec[k] for k in range(L)]


def _mlp_2in(x0, x1, w1s0, w1s1, b1s, w2s, b2v):
    dout = len(b2v)
    acc = [jnp.zeros((L,), F32) for _ in range(dout)]
    for k in range(16):
        t = jnp.maximum(x0 * w1s0[k] + x1 * w1s1[k] + b1s[k], 0.0)
        for j in range(dout):
            acc[j] = acc[j] + w2s[j][k] * t
    return [jnp.maximum(acc[j] + b2v[j], 0.0) for j in range(dout)]


def _eprep_body(eid_ref, eiu_ref, ad_ref, au_ref,
                edw1_ref, edb1_ref, edw2_ref, edb2_ref,
                euw1_ref, eub1_ref, euw2_ref, eub2_ref,
                dstd_ref, srcu_ref, dstu_ref, md_ref, h0_ref, h1_ref):
    dstd_ref[...] = eid_ref[1, :]
    srcu_ref[...] = eiu_ref[0, :]
    dstu_ref[...] = eiu_ref[1, :]
    a = ad_ref[0:1, :]
    h = jnp.maximum(jnp.dot(edw1_ref[...], a,
                            preferred_element_type=F32) + edb1_ref[...], 0.0)
    m = jnp.maximum(jnp.dot(edw2_ref[...], h,
                            preferred_element_type=F32) + edb2_ref[...], 0.0)
    md_ref[...] = m[0, :]
    b = au_ref[0:1, :]
    hu = jnp.maximum(jnp.dot(euw1_ref[...], b,
                             preferred_element_type=F32) + eub1_ref[...], 0.0)
    mu = jnp.maximum(jnp.dot(euw2_ref[...], hu,
                             preferred_element_type=F32) + eub2_ref[...], 0.0)
    h0_ref[...] = mu[0, :]
    h1_ref[...] = mu[1, :]


def _make_eprep(e):
    cb = 25600
    assert e % cb == 0
    grid = e // cb
    row2 = pl.BlockSpec((2, cb), lambda i: (0, i))
    flat = pl.BlockSpec((cb,), lambda i: (i,))

    def wspec(shape):
        return pl.BlockSpec(shape, lambda i: tuple(0 for _ in shape))

    return pl.pallas_call(
        _eprep_body,
        grid=(grid,),
        in_specs=[row2, row2, row2, row2,
                  wspec((16, 1)), wspec((16, 1)), wspec((1, 16)), wspec((1, 1)),
                  wspec((16, 1)), wspec((16, 1)), wspec((2, 16)), wspec((2, 1))],
        out_specs=[flat, flat, flat, flat, flat, flat],
        out_shape=[
            jax.ShapeDtypeStruct((e,), I32),
            jax.ShapeDtypeStruct((e,), I32),
            jax.ShapeDtypeStruct((e,), I32),
            jax.ShapeDtypeStruct((e,), F32),
            jax.ShapeDtypeStruct((e,), F32),
            jax.ShapeDtypeStruct((e,), F32),
        ],
    )


def _pass_body(ad_ref, au_ref, pd_ref, pu_ref):
    pd_ref[...] = ad_ref[...]
    pu_ref[...] = au_ref[...]


def _make_passthrough(e):
    cb = 25600
    grid = e // cb
    row2 = pl.BlockSpec((2, cb), lambda i: (0, i))
    return pl.pallas_call(
        _pass_body,
        grid=(grid,),
        in_specs=[row2, row2],
        out_specs=[row2, row2],
        out_shape=[
            jax.ShapeDtypeStruct((2, e), F32),
            jax.ShapeDtypeStruct((2, e), F32),
        ],
    )


def _nprep_body(xp_ref, mw1_ref, mb1_ref, mw2_ref, mb2_ref,
                g0_ref, g1_ref, x0_ref, x1_ref):
    x = xp_ref[...]
    h = jnp.maximum(jnp.dot(mw1_ref[...], x,
                            preferred_element_type=F32) + mb1_ref[...], 0.0)
    g = jnp.maximum(jnp.dot(mw2_ref[...], h,
                            preferred_element_type=F32) + mb2_ref[...], 0.0)
    g0_ref[...] = g[0, :]
    g1_ref[...] = g[1, :]
    x0_ref[...] = x[0, :]
    x1_ref[...] = x[1, :]


def _make_nprep(nuep):
    cb = 14336
    assert nuep % cb == 0
    grid = nuep // cb
    row2 = pl.BlockSpec((2, cb), lambda i: (0, i))
    flat = pl.BlockSpec((cb,), lambda i: (i,))

    def wspec(shape):
        return pl.BlockSpec(shape, lambda i: tuple(0 for _ in shape))

    return pl.pallas_call(
        _nprep_body,
        grid=(grid,),
        in_specs=[row2,
                  wspec((16, 2)), wspec((16, 1)), wspec((2, 16)), wspec((2, 1))],
        out_specs=[flat, flat, flat, flat],
        out_shape=[
            jax.ShapeDtypeStruct((nuep,), F32),
            jax.ShapeDtypeStruct((nuep,), F32),
            jax.ShapeDtypeStruct((nuep,), F32),
            jax.ShapeDtypeStruct((nuep,), F32),
        ],
    )


def _make_edge_kernel(n_ue, n_ap, e, nuep, napp):
    ts = e // SUB
    assert e % SUB == 0
    nch = (ts // NW) // NDS
    assert nch >= 2 and nch % 2 == 0
    gpt = nuep // NS
    apt = napp // NS

    mesh = plsc.VectorSubcoreMesh(core_axis_name="c", subcore_axis_name="s")

    @functools.partial(
        pl.kernel,
        out_type=(
            jax.ShapeDtypeStruct((NC, 2, nuep), F32),
            jax.ShapeDtypeStruct((NC, 3, napp), F32),
        ),
        mesh=mesh,
        compiler_params=pltpu.CompilerParams(
            use_tc_tiling_on_sc=False, needs_layout_passes=False),
        scratch_types=[
            pltpu.VMEM_SHARED((nuep,), F32),
            pltpu.VMEM_SHARED((nuep,), F32),
            pltpu.VMEM_SHARED((nuep,), F32),
            pltpu.VMEM_SHARED((nuep,), F32),
            pltpu.VMEM_SHARED((napp,), F32),
            pltpu.VMEM_SHARED((napp,), F32),
            pltpu.VMEM_SHARED((napp,), F32),
            pltpu.VMEM((gpt,), F32),
            pltpu.VMEM((2, NDS, SUB), I32),
            pltpu.VMEM((2, NDS, SUB), F32),
            pltpu.VMEM((NDS, SUB), F32),
            pltpu.VMEM((2, NDS, SUB), I32),
            pltpu.VMEM((2, CH), I32),
            pltpu.VMEM((2, CH), F32),
            pltpu.VMEM((2, CH), F32),
            pltpu.VMEM((NDS, SUB), F32),
            pltpu.VMEM((NDS, SUB), F32),
            pltpu.VMEM((napp,), F32),
            pltpu.VMEM((napp,), F32),
            pltpu.VMEM((napp,), F32),
            pltpu.VMEM((napp // SUB, SUB), I32),
            pltpu.SemaphoreType.DMA,
            pltpu.SemaphoreType.DMA,
            pltpu.SemaphoreType.DMA,
            pltpu.SemaphoreType.DMA,
            pltpu.SemaphoreType.DMA,
        ],
    )
    def edge_kernel(dstd2_hbm, md2_hbm, srcu2_hbm, dstu_hbm, h0_hbm, h1_hbm,
                    g0_hbm, g1_hbm, zue_hbm, zap_hbm, iotap_hbm, ones_hbm,
                    ue_parts, ap_parts,
                    g0_sp, g1_sp, ue_sum, ue_cnt, ap_s0, ap_s1, ap_cnt,
                    zb_v, dstb_v, msgd_v, ones_v,
                    srcb_v, dstu_v, h0_v, h1_v, gr0_v, gr1_v,
                    aps0_v, aps1_v, apc_v, iota_v,
                    sem_in0, sem_in1, sem_sc0, sem_sc1, sem_g):
        c = lax.axis_index("c")
        s = lax.axis_index("s")
        wid = c * NS + s
        sem_in = (sem_in0, sem_in1)
        sem_sc = (sem_sc0, sem_sc1)
        ones16 = jnp.ones((L,), F32)

        pltpu.sync_copy(ones_hbm, ones_v)
        pltpu.sync_copy(iotap_hbm, iota_v)

        pltpu.sync_copy(zue_hbm, zb_v)
        pltpu.sync_copy(zb_v, ue_sum.at[pl.ds(s * gpt, gpt)])
        pltpu.sync_copy(zb_v, ue_cnt.at[pl.ds(s * gpt, gpt)])
        pltpu.sync_copy(zb_v.at[pl.ds(0, apt)], ap_s0.at[pl.ds(s * apt, apt)])
        pltpu.sync_copy(zb_v.at[pl.ds(0, apt)], ap_s1.at[pl.ds(s * apt, apt)])
        pltpu.sync_copy(zb_v.at[pl.ds(0, apt)], ap_cnt.at[pl.ds(s * apt, apt)])
        pltpu.sync_copy(zap_hbm, aps0_v)
        pltpu.sync_copy(zap_hbm, aps1_v)
        pltpu.sync_copy(zap_hbm, apc_v)

        pltpu.sync_copy(g0_hbm.at[pl.ds(s * gpt, gpt)], zb_v)
        pltpu.sync_copy(zb_v, g0_sp.at[pl.ds(s * gpt, gpt)])
        pltpu.sync_copy(g1_hbm.at[pl.ds(s * gpt, gpt)], zb_v)
        pltpu.sync_copy(zb_v, g1_sp.at[pl.ds(s * gpt, gpt)])

        plsc.subcore_barrier()

        sub_base = (wid * ts) // NW
        nsubs = ((wid + 1) * ts) // NW - sub_base
        tail_subs = nsubs - nch * NDS

        def d_in_copies(j, slot):
            r = sub_base + j * NDS
            return (
                pltpu.make_async_copy(
                    dstd2_hbm.at[pl.ds(r, NDS), :], dstb_v.at[slot], sem_in[slot]),
                pltpu.make_async_copy(
                    md2_hbm.at[pl.ds(r, NDS), :], msgd_v.at[slot], sem_in[slot]),
            )

        def d_in_start(j, slot):
            for cp in d_in_copies(j, slot):
                cp.start()

        def d_in_wait(j, slot):
            for cp in d_in_copies(j, slot):
                cp.wait()

        def d_fire(slot):
            for k in range(NDS):
                idx = dstb_v.at[slot, k]
                pltpu.async_copy(msgd_v.at[slot, k], ue_sum.at[idx],
                                 sem_sc[slot], add=True)
                pltpu.async_copy(ones_v.at[k], ue_cnt.at[idx],
                                 sem_sc[slot], add=True)

        def d_drain(slot):
            for k in range(NDS):
                idx = dstb_v.at[slot, k]
                pltpu.make_async_copy(msgd_v.at[slot, k], ue_sum.at[idx],
                                      sem_sc[slot]).wait()
                pltpu.make_async_copy(ones_v.at[k], ue_cnt.at[idx],
                                      sem_sc[slot]).wait()

        d_in_start(0, 0)
        d_in_wait(0, 0)
        d_in_start(1, 1)
        d_fire(0)
        d_in_wait(1, 1)
        d_drain(0)
        d_in_start(2, 0)
        d_fire(1)

        def d_pair(jj, carry):
            j0 = 2 * jj
            j1 = j0 + 1
            d_in_wait(j0, 0)
            d_drain(1)
            d_in_start(j1, 1)
            d_fire(0)
            d_in_wait(j1, 1)
            d_drain(0)

            @pl.when(j1 + 1 < nch)
            def _():
                d_in_start(j1 + 1, 0)

            d_fire(1)
            return carry

        lax.fori_loop(1, nch // 2, d_pair, 0)
        d_drain(1)

        def d_tail(t, carry):
            r = sub_base + nch * NDS + t
            pltpu.sync_copy(dstd2_hbm.at[r], dstb_v.at[0, 0])
            pltpu.sync_copy(md2_hbm.at[r], msgd_v.at[0, 0])
            idx = dstb_v.at[0, 0]
            pltpu.sync_copy(msgd_v.at[0, 0], ue_sum.at[idx], add=True)
            pltpu.sync_copy(ones_v.at[0], ue_cnt.at[idx], add=True)
            return carry

        lax.fori_loop(0, tail_subs, d_tail, 0)

        def u_in_copies(j, slot):
            r = sub_base + j * NDS
            return (
                pltpu.make_async_copy(
                    srcu2_hbm.at[pl.ds(r, NDS), :], srcb_v.at[slot], sem_in[slot]),
                pltpu.make_async_copy(
                    dstu_hbm.at[pl.ds(r * SUB, CH)], dstu_v.at[slot], sem_in[slot]),
                pltpu.make_async_copy(
                    h0_hbm.at[pl.ds(r * SUB, CH)], h0_v.at[slot], sem_in[slot]),
                pltpu.make_async_copy(
                    h1_hbm.at[pl.ds(r * SUB, CH)], h1_v.at[slot], sem_in[slot]),
            )

        def u_in_start(j, slot):
            for cp in u_in_copies(j, slot):
                cp.start()

        def u_in_wait(j, slot):
            for cp in u_in_copies(j, slot):
                cp.wait()

        def u_gather_copies(slot):
            copies = []
            for k in range(NDS):
                idx = srcb_v.at[slot, k]
                copies.append(pltpu.make_async_copy(
                    g0_sp.at[idx], gr0_v.at[k], sem_g))
                copies.append(pltpu.make_async_copy(
                    g1_sp.at[idx], gr1_v.at[k], sem_g))
            return copies

        def u_combine(slot, k):
            def cgrp(gj, carry):
                sl = pl.ds(k * SUB + gj * L, L)
                slg = pl.ds(gj * L, L)
                dst16 = dstu_v[slot, sl]
                m0 = gr0_v[k, slg] + h0_v[slot, sl]
                m1 = gr1_v[k, slg] + h1_v[slot, sl]
                plsc.addupdate_scatter(aps0_v, [dst16], m0)
                plsc.addupdate_scatter(aps1_v, [dst16], m1)
                plsc.addupdate_scatter(apc_v, [dst16], ones16)
                return carry
            lax.fori_loop(0, SUB // L, cgrp, 0)

        def u_process(slot):
            for cp in u_gather_copies(slot):
                cp.start()
            for cp in u_gather_copies(slot):
                cp.wait()
            for k in range(NDS):
                u_combine(slot, k)

        u_in_start(0, 0)

        def u_pair(jj, carry):
            j0 = 2 * jj
            j1 = j0 + 1
            u_in_wait(j0, 0)
            u_in_start(j1, 1)
            u_process(0)
            u_in_wait(j1, 1)

            @pl.when(j1 + 1 < nch)
            def _():
                u_in_start(j1 + 1, 0)

            u_process(1)
            return carry

        lax.fori_loop(0, nch // 2, u_pair, 0)

        def u_tail(t, carry):
            r = sub_base + nch * NDS + t
            pltpu.sync_copy(srcu2_hbm.at[r], srcb_v.at[0, 0])
            pltpu.sync_copy(dstu_hbm.at[pl.ds(r * SUB, SUB)],
                            dstu_v.at[0, pl.ds(0, SUB)])
            pltpu.sync_copy(h0_hbm.at[pl.ds(r * SUB, SUB)],
                            h0_v.at[0, pl.ds(0, SUB)])
            pltpu.sync_copy(h1_hbm.at[pl.ds(r * SUB, SUB)],
                            h1_v.at[0, pl.ds(0, SUB)])
            idx = srcb_v.at[0, 0]
            pltpu.sync_copy(g0_sp.at[idx], gr0_v.at[0])
            pltpu.sync_copy(g1_sp.at[idx], gr1_v.at[0])
            u_combine(0, 0)
            return carry

        lax.fori_loop(0, tail_subs, u_tail, 0)

        for plane, hist in ((ap_s0, aps0_v), (ap_s1, aps1_v), (ap_cnt, apc_v)):
            for k in range(napp // SUB):
                pltpu.async_copy(
                    hist.at[pl.ds(k * SUB, SUB)],
                    plane.at[iota_v.at[k]], sem_g, add=True)
            for k in range(napp // SUB):
                pltpu.make_async_copy(
                    hist.at[pl.ds(k * SUB, SUB)],
                    plane.at[iota_v.at[k]], sem_g).wait()

        plsc.subcore_barrier()

        pltpu.sync_copy(ue_sum.at[pl.ds(s * gpt, gpt)], zb_v)
        pltpu.sync_copy(zb_v, ue_parts.at[c, 0, pl.ds(s * gpt, gpt)])
        pltpu.sync_copy(ue_cnt.at[pl.ds(s * gpt, gpt)], zb_v)
        pltpu.sync_copy(zb_v, ue_parts.at[c, 1, pl.ds(s * gpt, gpt)])
        pltpu.sync_copy(ap_s0.at[pl.ds(s * apt, apt)], zb_v.at[pl.ds(0, apt)])
        pltpu.sync_copy(zb_v.at[pl.ds(0, apt)], ap_parts.at[c, 0, pl.ds(s * apt, apt)])
        pltpu.sync_copy(ap_s1.at[pl.ds(s * apt, apt)], zb_v.at[pl.ds(0, apt)])
        pltpu.sync_copy(zb_v.at[pl.ds(0, apt)], ap_parts.at[c, 1, pl.ds(s * apt, apt)])
        pltpu.sync_copy(ap_cnt.at[pl.ds(s * apt, apt)], zb_v.at[pl.ds(0, apt)])
        pltpu.sync_copy(zb_v.at[pl.ds(0, apt)], ap_parts.at[c, 2, pl.ds(s * apt, apt)])

    return edge_kernel


def _make_finalize_kernel(nuep, napp):
    upt = nuep // NW
    apt = napp // NW
    mesh = plsc.VectorSubcoreMesh(core_axis_name="c", subcore_axis_name="s")

    @functools.partial(
        pl.kernel,
        out_type=(
            jax.ShapeDtypeStruct((nuep,), F32),
            jax.ShapeDtypeStruct((nuep,), F32),
            jax.ShapeDtypeStruct((napp,), F32),
            jax.ShapeDtypeStruct((napp,), F32),
        ),
        mesh=mesh,
        compiler_params=pltpu.CompilerParams(
            use_tc_tiling_on_sc=False, needs_layout_passes=False),
        scratch_types=[
            pltpu.VMEM((WP,), F32),
            pltpu.VMEM((nuep // NW,), F32),
            pltpu.VMEM((nuep // NW,), F32),
            pltpu.VMEM((nuep // NW,), F32),
            pltpu.VMEM((nuep // NW,), F32),
            pltpu.VMEM((nuep // NW,), F32),
            pltpu.VMEM((nuep // NW,), F32),
            pltpu.VMEM((nuep // NW,), F32),
            pltpu.VMEM((napp // NW,), F32),
            pltpu.VMEM((napp // NW,), F32),
            pltpu.VMEM((napp // NW,), F32),
            pltpu.VMEM((napp // NW,), F32),
            pltpu.VMEM((napp // NW,), F32),
            pltpu.VMEM((napp // NW,), F32),
            pltpu.VMEM((napp // NW,), F32),
            pltpu.VMEM((napp // NW,), F32),
        ],
    )
    def finalize_kernel(xp0_hbm, xp1_hbm, ue_parts, ap_parts, wts_hbm,
                        oue0_hbm, oue1_hbm, oap0_hbm, oap1_hbm,
                        wv, x0_v, x1_v, s0_v, s1_v, c0_v, c1_v, o1_v,
                        as00_v, as01_v, as10_v, as11_v, ac0_v, ac1_v,
                        oa0_v, oa1_v):
        c = lax.axis_index("c")
        s = lax.axis_index("s")
        wid = c * NS + s

        pltpu.sync_copy(wts_hbm, wv)

        ub = wid * upt
        pltpu.sync_copy(xp0_hbm.at[pl.ds(ub, upt)], x0_v)
        pltpu.sync_copy(xp1_hbm.at[pl.ds(ub, upt)], x1_v)
        pltpu.sync_copy(ue_parts.at[0, 0, pl.ds(ub, upt)], s0_v)
        pltpu.sync_copy(ue_parts.at[1, 0, pl.ds(ub, upt)], s1_v)
        pltpu.sync_copy(ue_parts.at[0, 1, pl.ds(ub, upt)], c0_v)
        pltpu.sync_copy(ue_parts.at[1, 1, pl.ds(ub, upt)], c1_v)

        upd_w1s0 = _scalars(_slot(wv, S_UPD_W1R0))
        upd_w1s1 = _scalars(_slot(wv, S_UPD_W1R1))
        upd_b1s = _scalars(_slot(wv, S_UPD_B1))
        upd_w2s = [_scalars(_slot(wv, S_UPD_W2C0))]
        upd_b2v = [_slot(wv, S_UPD_B2R0)]

        def ue_body(i, carry):
            sl = pl.ds(i * L, L)
            x0 = x0_v[sl]
            x1 = x1_v[sl]
            (r,) = _mlp_2in(x0, x1, upd_w1s0, upd_w1s1, upd_b1s, upd_w2s, upd_b2v)
            su = s0_v[sl] + s1_v[sl]
            cn = c0_v[sl] + c1_v[sl]
            avg = su / jnp.maximum(cn, 1.0)
            o1_v[sl] = avg + r
            return carry

        lax.fori_loop(0, upt // L, ue_body, 0)
        pltpu.sync_copy(x0_v, oue0_hbm.at[pl.ds(ub, upt)])
        pltpu.sync_copy(o1_v, oue1_hbm.at[pl.ds(ub, upt)])

        ab = wid * apt
        pltpu.sync_copy(ap_parts.at[0, 0, pl.ds(ab, apt)], as00_v)
        pltpu.sync_copy(ap_parts.at[1, 0, pl.ds(ab, apt)], as01_v)
        pltpu.sync_copy(ap_parts.at[0, 1, pl.ds(ab, apt)], as10_v)
        pltpu.sync_copy(ap_parts.at[1, 1, pl.ds(ab, apt)], as11_v)
        pltpu.sync_copy(ap_parts.at[0, 2, pl.ds(ab, apt)], ac0_v)
        pltpu.sync_copy(ap_parts.at[1, 2, pl.ds(ab, apt)], ac1_v)

        def ap_body(i, carry):
            sl = pl.ds(i * L, L)
            s0 = as00_v[sl] + as01_v[sl]
            s1 = as10_v[sl] + as11_v[sl]
            cn = ac0_v[sl] + ac1_v[sl]
            d = jnp.maximum(cn, 1.0)
            oa0_v[sl] = s0 / d
            oa1_v[sl] = s1 / d
            return carry

        lax.fori_loop(0, apt // L, ap_body, 0)
        pltpu.sync_copy(oa0_v, oap0_hbm.at[pl.ds(ab, apt)])
        pltpu.sync_copy(oa1_v, oap1_hbm.at[pl.ds(ab, apt)])

    return finalize_kernel


def _round_up(n, m):
    return (n + m - 1) // m * m


def kernel(x_ue, x_ap, edge_index_down, edge_attr_down, edge_index_up, edge_attr_up,
           upd_ue_w1, upd_ue_b1, upd_ue_w2, upd_ue_b2,
           msg_ue_w1, msg_ue_b1, msg_ue_w2, msg_ue_b2,
           edge_down_w1, edge_down_b1, edge_down_w2, edge_down_b2,
           edge_up_w1, edge_up_b1, edge_up_w2, edge_up_b2):
    n_ue = x_ue.shape[0]
    n_ap = x_ap.shape[0]
    e = edge_attr_down.shape[0]
    nuep = _round_up(n_ue + 1, NW * L)
    napp = _round_up(n_ap + 1, NW * L)

    dstd, srcu, dstu, md, h0, h1 = _make_eprep(e)(
        edge_index_down, edge_index_up,
        edge_attr_down.T, edge_attr_up.T,
        edge_down_w1.T, edge_down_b1[:, None], edge_down_w2.T,
        edge_down_b2[:, None],
        edge_up_w1.T, edge_up_b1[:, None], edge_up_w2.T, edge_up_b2[:, None])
    pd, pu = _make_passthrough(e)(edge_attr_down.T, edge_attr_up.T)

    xpT = jnp.pad(x_ue.T, ((0, 0), (0, nuep - n_ue)))
    g0, g1, xp0, xp1 = _make_nprep(nuep)(
        xpT, msg_ue_w1.T, msg_ue_b1[:, None], msg_ue_w2.T, msg_ue_b2[:, None])

    def rep(b):
        return jnp.full((L,), b, F32)

    wts = jnp.concatenate([
        upd_ue_w1[0], upd_ue_w1[1], upd_ue_b1, upd_ue_w2[:, 0], rep(upd_ue_b2[0]),
    ])

    zue = jnp.zeros((nuep // NS,), F32)
    zap = jnp.zeros((napp,), F32)
    iotap = jnp.arange(napp, dtype=I32).reshape(napp // SUB, SUB)
    ones = jnp.ones((NDS, SUB), F32)

    edge_kernel = _make_edge_kernel(n_ue, n_ap, e, nuep, napp)
    ue_parts, ap_parts = edge_kernel(
        dstd.reshape(e // SUB, SUB), md.reshape(e // SUB, SUB),
        srcu.reshape(e // SUB, SUB), dstu, h0, h1,
        g0, g1, zue, zap, iotap, ones)

    finalize_kernel = _make_finalize_kernel(nuep, napp)
    oue0, oue1, oap0, oap1 = finalize_kernel(xp0, xp1, ue_parts, ap_parts, wts)

    out_ue = jnp.stack([oue0[:n_ue], oue1[:n_ue]], axis=1)
    out_ap = jnp.stack([oap0[:n_ap], oap1[:n_ap]], axis=1)

    return out_ue, out_ap, pd.T, pu.T


# speedup vs baseline: 1.1597x; 1.1597x over previous
import functools

import jax
import jax.numpy as jnp
from jax import lax
from jax.experimental import pallas as pl
from jax.experimental.pallas import tpu as pltpu
from jax.experimental.pallas import tpu_sc as plsc

F32 = jnp.float32
I32 = jnp.int32

NC = 2
NS = 16
NW = NC * NS
L = 16
SUB = 128
NDS = 10
CH = NDS * SUB

S_UPD_W1R0, S_UPD_W1R1, S_UPD_B1, S_UPD_W2C0, S_UPD_B2R0 = 0, 1, 2, 3, 4
NSLOT = 5
WP = NSLOT * L


def _slot(wv, s):
    return wv[pl.ds(s * L, L)]


def _scalars(vec):
    return [vec[k] for k in range(L)]


def _mlp_2in(x0, x1, w1s0, w1s1, b1s, w2s, b2v):
    dout = len(b2v)
    acc = [jnp.zeros((L,), F32) for _ in range(dout)]
    for k in range(16):
        t = jnp.maximum(x0 * w1s0[k] + x1 * w1s1[k] + b1s[k], 0.0)
        for j in range(dout):
            acc[j] = acc[j] + w2s[j][k] * t
    return [jnp.maximum(acc[j] + b2v[j], 0.0) for j in range(dout)]


def _eprep_down_body(eid_ref, ad_ref,
                     edw1_ref, edb1_ref, edw2_ref, edb2_ref,
                     dstd_ref, md_ref):
    dstd_ref[...] = eid_ref[1, :]
    a = ad_ref[0:1, :]
    h = jnp.maximum(jnp.dot(edw1_ref[...], a,
                            preferred_element_type=F32) + edb1_ref[...], 0.0)
    m = jnp.maximum(jnp.dot(edw2_ref[...], h,
                            preferred_element_type=F32) + edb2_ref[...], 0.0)
    md_ref[...] = m[0, :]


def _eprep_up_body(eiu_ref, ad_ref, au_ref,
                   euw1_ref, eub1_ref, euw2_ref, eub2_ref,
                   srcu_ref, dstu_ref, h0_ref, h1_ref, pd_ref, pu_ref):
    srcu_ref[...] = eiu_ref[0, :]
    dstu_ref[...] = eiu_ref[1, :]
    b = au_ref[0:1, :]
    hu = jnp.maximum(jnp.dot(euw1_ref[...], b,
                             preferred_element_type=F32) + eub1_ref[...], 0.0)
    mu = jnp.maximum(jnp.dot(euw2_ref[...], hu,
                             preferred_element_type=F32) + eub2_ref[...], 0.0)
    h0_ref[...] = mu[0, :]
    h1_ref[...] = mu[1, :]
    pd_ref[...] = ad_ref[...]
    pu_ref[...] = au_ref[...]


def _wspec(shape):
    return pl.BlockSpec(shape, lambda i: tuple(0 for _ in shape))


def _make_eprep_down(e):
    cb = 25600
    assert e % cb == 0
    grid = e // cb
    row2 = pl.BlockSpec((2, cb), lambda i: (0, i))
    flat = pl.BlockSpec((cb,), lambda i: (i,))

    return pl.pallas_call(
        _eprep_down_body,
        grid=(grid,),
        in_specs=[row2, row2,
                  _wspec((16, 1)), _wspec((16, 1)), _wspec((1, 16)),
                  _wspec((1, 1))],
        out_specs=[flat, flat],
        out_shape=[
            jax.ShapeDtypeStruct((e,), I32),
            jax.ShapeDtypeStruct((e,), F32),
        ],
    )


def _make_eprep_up(e):
    cb = 25600
    grid = e // cb
    row2 = pl.BlockSpec((2, cb), lambda i: (0, i))
    flat = pl.BlockSpec((cb,), lambda i: (i,))

    return pl.pallas_call(
        _eprep_up_body,
        grid=(grid,),
        in_specs=[row2, row2, row2,
                  _wspec((16, 1)), _wspec((16, 1)), _wspec((2, 16)),
                  _wspec((2, 1))],
        out_specs=[flat, flat, flat, flat, row2, row2],
        out_shape=[
            jax.ShapeDtypeStruct((e,), I32),
            jax.ShapeDtypeStruct((e,), I32),
            jax.ShapeDtypeStruct((e,), F32),
            jax.ShapeDtypeStruct((e,), F32),
            jax.ShapeDtypeStruct((2, e), F32),
            jax.ShapeDtypeStruct((2, e), F32),
        ],
    )


def _nprep_body(xp_ref, mw1_ref, mb1_ref, mw2_ref, mb2_ref,
                g0_ref, g1_ref, x0_ref, x1_ref):
    x = xp_ref[...]
    h = jnp.maximum(jnp.dot(mw1_ref[...], x,
                            preferred_element_type=F32) + mb1_ref[...], 0.0)
    g = jnp.maximum(jnp.dot(mw2_ref[...], h,
                            preferred_element_type=F32) + mb2_ref[...], 0.0)
    g0_ref[...] = g[0, :]
    g1_ref[...] = g[1, :]
    x0_ref[...] = x[0, :]
    x1_ref[...] = x[1, :]


def _make_nprep(nuep):
    cb = 14336
    assert nuep % cb == 0
    grid = nuep // cb
    row2 = pl.BlockSpec((2, cb), lambda i: (0, i))
    flat = pl.BlockSpec((cb,), lambda i: (i,))

    def wspec(shape):
        return pl.BlockSpec(shape, lambda i: tuple(0 for _ in shape))

    return pl.pallas_call(
        _nprep_body,
        grid=(grid,),
        in_specs=[row2,
                  wspec((16, 2)), wspec((16, 1)), wspec((2, 16)), wspec((2, 1))],
        out_specs=[flat, flat, flat, flat],
        out_shape=[
            jax.ShapeDtypeStruct((nuep,), F32),
            jax.ShapeDtypeStruct((nuep,), F32),
            jax.ShapeDtypeStruct((nuep,), F32),
            jax.ShapeDtypeStruct((nuep,), F32),
        ],
    )


def _make_edge_kernel(n_ue, n_ap, e, nuep, napp):
    ts = e // SUB
    assert e % SUB == 0
    nch = (ts // NW) // NDS
    assert nch >= 2 and nch % 2 == 0
    gpt = nuep // NS
    apt = napp // NS

    mesh = plsc.VectorSubcoreMesh(core_axis_name="c", subcore_axis_name="s")

    @functools.partial(
        pl.kernel,
        out_type=jax.ShapeDtypeStruct((NC, 2, nuep), F32),
        mesh=mesh,
        compiler_params=pltpu.CompilerParams(
            use_tc_tiling_on_sc=False, needs_layout_passes=False),
        scratch_types=[
            pltpu.VMEM_SHARED((nuep,), F32),
            pltpu.VMEM_SHARED((nuep,), F32),
            pltpu.VMEM((gpt,), F32),
            pltpu.VMEM((2, NDS, SUB), I32),
            pltpu.VMEM((2, NDS, SUB), F32),
            pltpu.VMEM((NDS, SUB), F32),
            pltpu.SemaphoreType.DMA,
            pltpu.SemaphoreType.DMA,
            pltpu.SemaphoreType.DMA,
            pltpu.SemaphoreType.DMA,
        ],
    )
    def down_kernel(dstd2_hbm, md2_hbm, zue_hbm, ones_hbm,
                    ue_parts,
                    ue_sum, ue_cnt,
                    zb_v, dstb_v, msgd_v, ones_v,
                    sem_in0, sem_in1, sem_sc0, sem_sc1):
        c = lax.axis_index("c")
        s = lax.axis_index("s")
        wid = c * NS + s
        sem_in = (sem_in0, sem_in1)
        sem_sc = (sem_sc0, sem_sc1)

        pltpu.sync_copy(ones_hbm, ones_v)

        pltpu.sync_copy(zue_hbm, zb_v)
        pltpu.sync_copy(zb_v, ue_sum.at[pl.ds(s * gpt, gpt)])
        pltpu.sync_copy(zb_v, ue_cnt.at[pl.ds(s * gpt, gpt)])

        plsc.subcore_barrier()

        sub_base = (wid * ts) // NW
        nsubs = ((wid + 1) * ts) // NW - sub_base
        tail_subs = nsubs - nch * NDS

        def d_in_copies(j, slot):
            r = sub_base + j * NDS
            return (
                pltpu.make_async_copy(
                    dstd2_hbm.at[pl.ds(r, NDS), :], dstb_v.at[slot], sem_in[slot]),
                pltpu.make_async_copy(
                    md2_hbm.at[pl.ds(r, NDS), :], msgd_v.at[slot], sem_in[slot]),
            )

        def d_in_start(j, slot):
            for cp in d_in_copies(j, slot):
                cp.start()

        def d_in_wait(j, slot):
            for cp in d_in_copies(j, slot):
                cp.wait()

        def d_fire(slot):
            for k in range(NDS):
                idx = dstb_v.at[slot, k]
                pltpu.async_copy(msgd_v.at[slot, k], ue_sum.at[idx],
                                 sem_sc[slot], add=True)
                pltpu.async_copy(ones_v.at[k], ue_cnt.at[idx],
                                 sem_sc[slot], add=True)

        def d_drain(slot):
            for k in range(NDS):
                idx = dstb_v.at[slot, k]
                pltpu.make_async_copy(msgd_v.at[slot, k], ue_sum.at[idx],
                                      sem_sc[slot]).wait()
                pltpu.make_async_copy(ones_v.at[k], ue_cnt.at[idx],
                                      sem_sc[slot]).wait()

        d_in_start(0, 0)
        d_in_wait(0, 0)
        d_in_start(1, 1)
        d_fire(0)
        d_in_wait(1, 1)
        d_drain(0)
        d_in_start(2, 0)
        d_fire(1)

        def d_pair(jj, carry):
            j0 = 2 * jj
            j1 = j0 + 1
            d_in_wait(j0, 0)
            d_drain(1)
            d_in_start(j1, 1)
            d_fire(0)
            d_in_wait(j1, 1)
            d_drain(0)

            @pl.when(j1 + 1 < nch)
            def _():
                d_in_start(j1 + 1, 0)

            d_fire(1)
            return carry

        lax.fori_loop(1, nch // 2, d_pair, 0)
        d_drain(1)

        def d_tail(t, carry):
            r = sub_base + nch * NDS + t
            pltpu.sync_copy(dstd2_hbm.at[r], dstb_v.at[0, 0])
            pltpu.sync_copy(md2_hbm.at[r], msgd_v.at[0, 0])
            idx = dstb_v.at[0, 0]
            pltpu.sync_copy(msgd_v.at[0, 0], ue_sum.at[idx], add=True)
            pltpu.sync_copy(ones_v.at[0], ue_cnt.at[idx], add=True)
            return carry

        lax.fori_loop(0, tail_subs, d_tail, 0)

        plsc.subcore_barrier()

        pltpu.sync_copy(ue_sum.at[pl.ds(s * gpt, gpt)], zb_v)
        pltpu.sync_copy(zb_v, ue_parts.at[c, 0, pl.ds(s * gpt, gpt)])
        pltpu.sync_copy(ue_cnt.at[pl.ds(s * gpt, gpt)], zb_v)
        pltpu.sync_copy(zb_v, ue_parts.at[c, 1, pl.ds(s * gpt, gpt)])

    @functools.partial(
        pl.kernel,
        out_type=jax.ShapeDtypeStruct((NC, 3, napp), F32),
        mesh=mesh,
        compiler_params=pltpu.CompilerParams(
            use_tc_tiling_on_sc=False, needs_layout_passes=False),
        scratch_types=[
            pltpu.VMEM_SHARED((nuep,), F32),
            pltpu.VMEM_SHARED((nuep,), F32),
            pltpu.VMEM_SHARED((napp,), F32),
            pltpu.VMEM_SHARED((napp,), F32),
            pltpu.VMEM_SHARED((napp,), F32),
            pltpu.VMEM((gpt,), F32),
            pltpu.VMEM((2, NDS, SUB), I32),
            pltpu.VMEM((2, CH), I32),
            pltpu.VMEM((2, CH), F32),
            pltpu.VMEM((2, CH), F32),
            pltpu.VMEM((NDS, SUB), F32),
            pltpu.VMEM((NDS, SUB), F32),
            pltpu.VMEM((napp,), F32),
            pltpu.VMEM((napp,), F32),
            pltpu.VMEM((napp,), F32),
            pltpu.VMEM((napp // SUB, SUB), I32),
            pltpu.SemaphoreType.DMA,
            pltpu.SemaphoreType.DMA,
            pltpu.SemaphoreType.DMA,
        ],
    )
    def up_kernel(srcu2_hbm, dstu_hbm, h0_hbm, h1_hbm,
                  g0_hbm, g1_hbm, zue_hbm, zap_hbm, iotap_hbm,
                  ap_parts,
                  g0_sp, g1_sp, ap_s0, ap_s1, ap_cnt,
                  zb_v, srcb_v, dstu_v, h0_v, h1_v, gr0_v, gr1_v,
                  aps0_v, aps1_v, apc_v, iota_v,
                  sem_in0, sem_in1, sem_g):
        c = lax.axis_index("c")
        s = lax.axis_index("s")
        wid = c * NS + s
        sem_in = (sem_in0, sem_in1)
        ones16 = jnp.ones((L,), F32)

        pltpu.sync_copy(iotap_hbm, iota_v)

        pltpu.sync_copy(zue_hbm, zb_v)
        pltpu.sync_copy(zb_v.at[pl.ds(0, apt)], ap_s0.at[pl.ds(s * apt, apt)])
        pltpu.sync_copy(zb_v.at[pl.ds(0, apt)], ap_s1.at[pl.ds(s * apt, apt)])
        pltpu.sync_copy(zb_v.at[pl.ds(0, apt)], ap_cnt.at[pl.ds(s * apt, apt)])
        pltpu.sync_copy(zap_hbm, aps0_v)
        pltpu.sync_copy(zap_hbm, aps1_v)
        pltpu.sync_copy(zap_hbm, apc_v)

        pltpu.sync_copy(g0_hbm.at[pl.ds(s * gpt, gpt)], zb_v)
        pltpu.sync_copy(zb_v, g0_sp.at[pl.ds(s * gpt, gpt)])
        pltpu.sync_copy(g1_hbm.at[pl.ds(s * gpt, gpt)], zb_v)
        pltpu.sync_copy(zb_v, g1_sp.at[pl.ds(s * gpt, gpt)])

        plsc.subcore_barrier()

        sub_base = (wid * ts) // NW
        nsubs = ((wid + 1) * ts) // NW - sub_base
        tail_subs = nsubs - nch * NDS

        def u_in_copies(j, slot):
            r = sub_base + j * NDS
            return (
                pltpu.make_async_copy(
                    srcu2_hbm.at[pl.ds(r, NDS), :], srcb_v.at[slot], sem_in[slot]),
                pltpu.make_async_copy(
                    dstu_hbm.at[pl.ds(r * SUB, CH)], dstu_v.at[slot], sem_in[slot]),
                pltpu.make_async_copy(
                    h0_hbm.at[pl.ds(r * SUB, CH)], h0_v.at[slot], sem_in[slot]),
                pltpu.make_async_copy(
                    h1_hbm.at[pl.ds(r * SUB, CH)], h1_v.at[slot], sem_in[slot]),
            )

        def u_in_start(j, slot):
            for cp in u_in_copies(j, slot):
                cp.start()

        def u_in_wait(j, slot):
            for cp in u_in_copies(j, slot):
                cp.wait()

        def u_gather_copies(slot):
            copies = []
            for k in range(NDS):
                idx = srcb_v.at[slot, k]
                copies.append(pltpu.make_async_copy(
                    g0_sp.at[idx], gr0_v.at[k], sem_g))
                copies.append(pltpu.make_async_copy(
                    g1_sp.at[idx], gr1_v.at[k], sem_g))
            return copies

        def u_combine(slot, k):
            def cgrp(gj, carry):
                sl = pl.ds(k * SUB + gj * L, L)
                slg = pl.ds(gj * L, L)
                dst16 = dstu_v[slot, sl]
                m0 = gr0_v[k, slg] + h0_v[slot, sl]
                m1 = gr1_v[k, slg] + h1_v[slot, sl]
                plsc.addupdate_scatter(aps0_v, [dst16], m0)
                plsc.addupdate_scatter(aps1_v, [dst16], m1)
                plsc.addupdate_scatter(apc_v, [dst16], ones16)
                return carry
            lax.fori_loop(0, SUB // L, cgrp, 0)

        def u_process(slot):
            for cp in u_gather_copies(slot):
                cp.start()
            for cp in u_gather_copies(slot):
                cp.wait()
            for k in range(NDS):
                u_combine(slot, k)

        u_in_start(0, 0)

        def u_pair(jj, carry):
            j0 = 2 * jj
            j1 = j0 + 1
            u_in_wait(j0, 0)
            u_in_start(j1, 1)
            u_process(0)
            u_in_wait(j1, 1)

            @pl.when(j1 + 1 < nch)
            def _():
                u_in_start(j1 + 1, 0)

            u_process(1)
            return carry

        lax.fori_loop(0, nch // 2, u_pair, 0)

        def u_tail(t, carry):
            r = sub_base + nch * NDS + t
            pltpu.sync_copy(srcu2_hbm.at[r], srcb_v.at[0, 0])
            pltpu.sync_copy(dstu_hbm.at[pl.ds(r * SUB, SUB)],
                            dstu_v.at[0, pl.ds(0, SUB)])
            pltpu.sync_copy(h0_hbm.at[pl.ds(r * SUB, SUB)],
                            h0_v.at[0, pl.ds(0, SUB)])
            pltpu.sync_copy(h1_hbm.at[pl.ds(r * SUB, SUB)],
                            h1_v.at[0, pl.ds(0, SUB)])
            idx = srcb_v.at[0, 0]
            pltpu.sync_copy(g0_sp.at[idx], gr0_v.at[0])
            pltpu.sync_copy(g1_sp.at[idx], gr1_v.at[0])
            u_combine(0, 0)
            return carry

        lax.fori_loop(0, tail_subs, u_tail, 0)

        for plane, hist in ((ap_s0, aps0_v), (ap_s1, aps1_v), (ap_cnt, apc_v)):
            for k in range(napp // SUB):
                pltpu.async_copy(
                    hist.at[pl.ds(k * SUB, SUB)],
                    plane.at[iota_v.at[k]], sem_g, add=True)
            for k in range(napp // SUB):
                pltpu.make_async_copy(
                    hist.at[pl.ds(k * SUB, SUB)],
                    plane.at[iota_v.at[k]], sem_g).wait()

        plsc.subcore_barrier()

        pltpu.sync_copy(ap_s0.at[pl.ds(s * apt, apt)], zb_v.at[pl.ds(0, apt)])
        pltpu.sync_copy(zb_v.at[pl.ds(0, apt)], ap_parts.at[c, 0, pl.ds(s * apt, apt)])
        pltpu.sync_copy(ap_s1.at[pl.ds(s * apt, apt)], zb_v.at[pl.ds(0, apt)])
        pltpu.sync_copy(zb_v.at[pl.ds(0, apt)], ap_parts.at[c, 1, pl.ds(s * apt, apt)])
        pltpu.sync_copy(ap_cnt.at[pl.ds(s * apt, apt)], zb_v.at[pl.ds(0, apt)])
        pltpu.sync_copy(zb_v.at[pl.ds(0, apt)], ap_parts.at[c, 2, pl.ds(s * apt, apt)])

    return down_kernel, up_kernel


def _make_finalize_kernel(nuep, napp):
    upt = nuep // NW
    apt = napp // NW
    mesh = plsc.VectorSubcoreMesh(core_axis_name="c", subcore_axis_name="s")

    @functools.partial(
        pl.kernel,
        out_type=(
            jax.ShapeDtypeStruct((nuep,), F32),
            jax.ShapeDtypeStruct((nuep,), F32),
            jax.ShapeDtypeStruct((napp,), F32),
            jax.ShapeDtypeStruct((napp,), F32),
        ),
        mesh=mesh,
        compiler_params=pltpu.CompilerParams(
            use_tc_tiling_on_sc=False, needs_layout_passes=False),
        scratch_types=[
            pltpu.VMEM((WP,), F32),
            pltpu.VMEM((nuep // NW,), F32),
            pltpu.VMEM((nuep // NW,), F32),
            pltpu.VMEM((nuep // NW,), F32),
            pltpu.VMEM((nuep // NW,), F32),
            pltpu.VMEM((nuep // NW,), F32),
            pltpu.VMEM((nuep // NW,), F32),
            pltpu.VMEM((nuep // NW,), F32),
            pltpu.VMEM((napp // NW,), F32),
            pltpu.VMEM((napp // NW,), F32),
            pltpu.VMEM((napp // NW,), F32),
            pltpu.VMEM((napp // NW,), F32),
            pltpu.VMEM((napp // NW,), F32),
            pltpu.VMEM((napp // NW,), F32),
            pltpu.VMEM((napp // NW,), F32),
            pltpu.VMEM((napp // NW,), F32),
        ],
    )
    def finalize_kernel(xp0_hbm, xp1_hbm, ue_parts, ap_parts, wts_hbm,
                        oue0_hbm, oue1_hbm, oap0_hbm, oap1_hbm,
                        wv, x0_v, x1_v, s0_v, s1_v, c0_v, c1_v, o1_v,
                        as00_v, as01_v, as10_v, as11_v, ac0_v, ac1_v,
                        oa0_v, oa1_v):
        c = lax.axis_index("c")
        s = lax.axis_index("s")
        wid = c * NS + s

        pltpu.sync_copy(wts_hbm, wv)

        ub = wid * upt
        pltpu.sync_copy(xp0_hbm.at[pl.ds(ub, upt)], x0_v)
        pltpu.sync_copy(xp1_hbm.at[pl.ds(ub, upt)], x1_v)
        pltpu.sync_copy(ue_parts.at[0, 0, pl.ds(ub, upt)], s0_v)
        pltpu.sync_copy(ue_parts.at[1, 0, pl.ds(ub, upt)], s1_v)
        pltpu.sync_copy(ue_parts.at[0, 1, pl.ds(ub, upt)], c0_v)
        pltpu.sync_copy(ue_parts.at[1, 1, pl.ds(ub, upt)], c1_v)

        upd_w1s0 = _scalars(_slot(wv, S_UPD_W1R0))
        upd_w1s1 = _scalars(_slot(wv, S_UPD_W1R1))
        upd_b1s = _scalars(_slot(wv, S_UPD_B1))
        upd_w2s = [_scalars(_slot(wv, S_UPD_W2C0))]
        upd_b2v = [_slot(wv, S_UPD_B2R0)]

        def ue_body(i, carry):
            sl = pl.ds(i * L, L)
            x0 = x0_v[sl]
            x1 = x1_v[sl]
            (r,) = _mlp_2in(x0, x1, upd_w1s0, upd_w1s1, upd_b1s, upd_w2s, upd_b2v)
            su = s0_v[sl] + s1_v[sl]
            cn = c0_v[sl] + c1_v[sl]
            avg = su / jnp.maximum(cn, 1.0)
            o1_v[sl] = avg + r
            return carry

        lax.fori_loop(0, upt // L, ue_body, 0)
        pltpu.sync_copy(x0_v, oue0_hbm.at[pl.ds(ub, upt)])
        pltpu.sync_copy(o1_v, oue1_hbm.at[pl.ds(ub, upt)])

        ab = wid * apt
        pltpu.sync_copy(ap_parts.at[0, 0, pl.ds(ab, apt)], as00_v)
        pltpu.sync_copy(ap_parts.at[1, 0, pl.ds(ab, apt)], as01_v)
        pltpu.sync_copy(ap_parts.at[0, 1, pl.ds(ab, apt)], as10_v)
        pltpu.sync_copy(ap_parts.at[1, 1, pl.ds(ab, apt)], as11_v)
        pltpu.sync_copy(ap_parts.at[0, 2, pl.ds(ab, apt)], ac0_v)
        pltpu.sync_copy(ap_parts.at[1, 2, pl.ds(ab, apt)], ac1_v)

        def ap_body(i, carry):
            sl = pl.ds(i * L, L)
            s0 = as00_v[sl] + as01_v[sl]
            s1 = as10_v[sl] + as11_v[sl]
            cn = ac0_v[sl] + ac1_v[sl]
            d = jnp.maximum(cn, 1.0)
            oa0_v[sl] = s0 / d
            oa1_v[sl] = s1 / d
            return carry

        lax.fori_loop(0, apt // L, ap_body, 0)
        pltpu.sync_copy(oa0_v, oap0_hbm.at[pl.ds(ab, apt)])
        pltpu.sync_copy(oa1_v, oap1_hbm.at[pl.ds(ab, apt)])

    return finalize_kernel


def _round_up(n, m):
    return (n + m - 1) // m * m


def kernel(x_ue, x_ap, edge_index_down, edge_attr_down, edge_index_up, edge_attr_up,
           upd_ue_w1, upd_ue_b1, upd_ue_w2, upd_ue_b2,
           msg_ue_w1, msg_ue_b1, msg_ue_w2, msg_ue_b2,
           edge_down_w1, edge_down_b1, edge_down_w2, edge_down_b2,
           edge_up_w1, edge_up_b1, edge_up_w2, edge_up_b2):
    n_ue = x_ue.shape[0]
    n_ap = x_ap.shape[0]
    e = edge_attr_down.shape[0]
    nuep = _round_up(n_ue + 1, NW * L)
    napp = _round_up(n_ap + 1, NW * L)

    dstd, md = _make_eprep_down(e)(
        edge_index_down, edge_attr_down.T,
        edge_down_w1.T, edge_down_b1[:, None], edge_down_w2.T,
        edge_down_b2[:, None])
    srcu, dstu, h0, h1, pd, pu = _make_eprep_up(e)(
        edge_index_up, edge_attr_down.T, edge_attr_up.T,
        edge_up_w1.T, edge_up_b1[:, None], edge_up_w2.T, edge_up_b2[:, None])

    xpT = jnp.pad(x_ue.T, ((0, 0), (0, nuep - n_ue)))
    g0, g1, xp0, xp1 = _make_nprep(nuep)(
        xpT, msg_ue_w1.T, msg_ue_b1[:, None], msg_ue_w2.T, msg_ue_b2[:, None])

    def rep(b):
        return jnp.full((L,), b, F32)

    wts = jnp.concatenate([
        upd_ue_w1[0], upd_ue_w1[1], upd_ue_b1, upd_ue_w2[:, 0], rep(upd_ue_b2[0]),
    ])

    zue = jnp.zeros((nuep // NS,), F32)
    zap = jnp.zeros((napp,), F32)
    iotap = jnp.arange(napp, dtype=I32).reshape(napp // SUB, SUB)
    ones = jnp.ones((NDS, SUB), F32)

    down_kernel, up_kernel = _make_edge_kernel(n_ue, n_ap, e, nuep, napp)
    ue_parts = down_kernel(
        dstd.reshape(e // SUB, SUB), md.reshape(e // SUB, SUB), zue, ones)
    ap_parts = up_kernel(
        srcu.reshape(e // SUB, SUB), dstu, h0, h1, g0, g1, zue, zap, iotap)

    finalize_kernel = _make_finalize_kernel(nuep, napp)
    oue0, oue1, oap0, oap1 = finalize_kernel(xp0, xp1, ue_parts, ap_parts, wts)

    out_ue = jnp.stack([oue0[:n_ue], oue1[:n_ue]], axis=1)
    out_ap = jnp.stack([oap0[:n_ap], oap1[:n_ap]], axis=1)

    return out_ue, out_ap, pd.T, pu.T
